# baseline (speedup 1.0000x reference)
import functools

import jax
import jax.numpy as jnp
from jax import lax
from jax.experimental import pallas as pl
from jax.experimental.pallas import tpu as pltpu
from jax.experimental.pallas import tpu_sc as plsc

N = 10000
E = 320000
D_IN = 128
D_H = 32
D_OUT = 64

NC = 2
NS = 16
NW = NC * NS
L = 16

CHUNK = 128
NBUF = 8
K = -(-E // (NW * CHUNK * NBUF)) * NBUF
K2 = K * NC
TOTC = NW * K
EPAD = TOTC * CHUNK
NPAD = 10240
RPT = NPAD // NS
ZR = 128

_MESH = plsc.VectorSubcoreMesh(
    core_axis_name="c", subcore_axis_name="s", num_cores=NC, num_subcores=NS)


@functools.partial(
    pl.kernel,
    out_type=jax.ShapeDtypeStruct((NC, NPAD), jnp.float32),
    mesh=_MESH,
    scratch_types=[
        pltpu.VMEM((K, CHUNK), jnp.int32),
        pltpu.VMEM((CHUNK,), jnp.float32),
        pltpu.VMEM((RPT,), jnp.float32),
        pltpu.VMEM_SHARED((NPAD,), jnp.float32),
    ],
)
def _deg_kernel(dst_hbm, out_hbm, dst_v, ones_v, zrow_v, acc):
    c = lax.axis_index("c")
    s = lax.axis_index("s")
    wid = c * NS + s
    for i in range(CHUNK // L):
        ones_v[pl.ds(i * L, L)] = jnp.ones((L,), jnp.float32)
    for i in range(RPT // L):
        zrow_v[pl.ds(i * L, L)] = jnp.zeros((L,), jnp.float32)
    pltpu.sync_copy(zrow_v, acc.at[pl.ds(s * RPT, RPT)])
    pltpu.sync_copy(dst_hbm.at[pl.ds(wid * K, K)], dst_v)
    plsc.subcore_barrier()

    def body(j, carry):
        pltpu.sync_copy(ones_v, acc.at[dst_v.at[j]], add=True)
        return carry

    lax.fori_loop(0, K, body, 0)
    plsc.subcore_barrier()
    pltpu.sync_copy(acc.at[pl.ds(s * RPT, RPT)],
                    out_hbm.at[c, pl.ds(s * RPT, RPT)])


def _rsqrt16(x):
    i = lax.bitcast_convert_type(x, jnp.int32)
    i = jnp.int32(0x5F3759DF) - lax.shift_right_arithmetic(i, 1)
    y = lax.bitcast_convert_type(i, jnp.float32)
    for _ in range(3):
        y = y * (1.5 - 0.5 * x * y * y)
    return y


def _make_scatter_kernel(D):
    @functools.partial(
        pl.kernel,
        out_type=jax.ShapeDtypeStruct((NC, NPAD, D), jnp.float32),
        mesh=_MESH,
        compiler_params=pltpu.CompilerParams(use_tc_tiling_on_sc=False,
                                             needs_layout_passes=False),
        scratch_types=[
            pltpu.VMEM((K, CHUNK), jnp.int32),
            pltpu.VMEM((K, CHUNK), jnp.int32),
            pltpu.VMEM((NBUF, CHUNK, D), jnp.float32),
            pltpu.VMEM((ZR, D), jnp.float32),
            pltpu.VMEM((RPT, D), jnp.float32),
            pltpu.VMEM((RPT,), jnp.float32),
            pltpu.VMEM((RPT,), jnp.float32),
            pltpu.VMEM_SHARED((NPAD, D), jnp.float32),
            pltpu.VMEM_SHARED((NPAD, D), jnp.float32),
        ] + [pltpu.SemaphoreType.DMA] * NBUF,
    )
    def _scatter(g_hbm, degp_hbm, src_hbm, dst_hbm, out_hbm,
                 src_v, dst_v, rows_v, zb_v, hbuf, degv, disv,
                 acc, g_sh, *sems):
        c = lax.axis_index("c")
        s = lax.axis_index("s")
        wid = c * NS + s

        def zfill(i, carry):
            zrow = zb_v.at[i]
            for dj in range(D // L):
                zrow[pl.ds(dj * L, L)] = jnp.zeros((L,), jnp.float32)
            return carry

        lax.fori_loop(0, ZR, zfill, 0)
        for t in range(RPT // ZR):
            pltpu.sync_copy(zb_v, acc.at[pl.ds(s * RPT + t * ZR, ZR)])
        pltpu.sync_copy(g_hbm.at[pl.ds(s * RPT, RPT)], hbuf)
        pltpu.sync_copy(degp_hbm.at[0, pl.ds(s * RPT, RPT)], degv)
        pltpu.sync_copy(degp_hbm.at[1, pl.ds(s * RPT, RPT)], disv)

        def disgrp(i, carry):
            d = degv[pl.ds(i * L, L)] + disv[pl.ds(i * L, L)] + 1.0
            disv[pl.ds(i * L, L)] = _rsqrt16(d)
            return carry

        lax.fori_loop(0, RPT // L, disgrp, 0)

        def rowscale(r, carry):
            sc = plsc.load_gather(disv, [jnp.zeros((L,), jnp.int32) + r])
            row = hbuf.at[r]
            for dj in range(D // L):
                row[pl.ds(dj * L, L)] = row[pl.ds(dj * L, L)] * sc
            return carry

        lax.fori_loop(0, RPT, rowscale, 0)
        pltpu.sync_copy(hbuf, g_sh.at[pl.ds(s * RPT, RPT)])
        pltpu.sync_copy(src_hbm.at[pl.ds(wid * K, K)], src_v)
        pltpu.sync_copy(dst_hbm.at[pl.ds(wid * K, K)], dst_v)
        plsc.subcore_barrier()

        for b in range(NBUF - 1):
            pltpu.async_copy(g_sh.at[src_v.at[b]], rows_v.at[b], sems[b])

        def body(i, carry):
            j0 = i * NBUF
            for bi in range(NBUF):
                j = j0 + bi
                jl = j + NBUF - 1
                bl = (bi + NBUF - 1) % NBUF

                @pl.when(jl < K)
                def _():
                    pltpu.async_copy(g_sh.at[src_v.at[jl]],
                                     rows_v.at[bl], sems[bl])

                pltpu.make_async_copy(g_sh.at[src_v.at[j]],
                                      rows_v.at[bi], sems[bi]).wait()
                pltpu.sync_copy(rows_v.at[bi], acc.at[dst_v.at[j]], add=True)
            return carry

        lax.fori_loop(0, K // NBUF, body, 0)
        plsc.subcore_barrier()
        pltpu.sync_copy(acc.at[pl.ds(s * RPT, RPT)],
                        out_hbm.at[c, pl.ds(s * RPT, RPT)])

    return _scatter


_scatter32 = _make_scatter_kernel(D_H)

DHALF = D_OUT // NC


@functools.partial(
    pl.kernel,
    out_type=jax.ShapeDtypeStruct((NC, NPAD, DHALF), jnp.float32),
    mesh=_MESH,
    compiler_params=pltpu.CompilerParams(use_tc_tiling_on_sc=False),
    scratch_types=[
        pltpu.VMEM((K2, CHUNK), jnp.int32),
        pltpu.VMEM((K2, CHUNK), jnp.int32),
        pltpu.VMEM((NBUF, CHUNK, DHALF), jnp.float32),
        pltpu.VMEM((ZR, DHALF), jnp.float32),
        pltpu.VMEM_SHARED((NPAD, DHALF), jnp.float32),
        pltpu.VMEM_SHARED((NPAD, DHALF), jnp.float32),
    ] + [pltpu.SemaphoreType.DMA] * NBUF,
)
def _scatter64f(g_hbm, src_hbm, dst_hbm, out_hbm,
                src_v, dst_v, rows_v, zb_v, acc, g_sh, *sems):
    c = lax.axis_index("c")
    s = lax.axis_index("s")

    def zfill(i, carry):
        for dj in range(DHALF // L):
            zb_v[i, pl.ds(dj * L, L)] = jnp.zeros((L,), jnp.float32)
        return carry

    lax.fori_loop(0, ZR, zfill, 0)
    for t in range(RPT // ZR):
        pltpu.sync_copy(zb_v, acc.at[pl.ds(s * RPT + t * ZR, ZR)])
    pltpu.sync_copy(g_hbm.at[c, pl.ds(s * RPT, RPT)],
                    g_sh.at[pl.ds(s * RPT, RPT)])
    pltpu.sync_copy(src_hbm.at[pl.ds(s * K2, K2)], src_v)
    pltpu.sync_copy(dst_hbm.at[pl.ds(s * K2, K2)], dst_v)
    plsc.subcore_barrier()

    for b in range(NBUF - 1):
        pltpu.async_copy(g_sh.at[src_v.at[b]], rows_v.at[b], sems[b])

    def body(i, carry):
        j0 = i * NBUF
        for bi in range(NBUF):
            j = j0 + bi
            jl = j + NBUF - 1
            bl = (bi + NBUF - 1) % NBUF

            @pl.when(jl < K2)
            def _():
                pltpu.async_copy(g_sh.at[src_v.at[jl]],
                                 rows_v.at[bl], sems[bl])

            pltpu.make_async_copy(g_sh.at[src_v.at[j]],
                                  rows_v.at[bi], sems[bi]).wait()
            pltpu.sync_copy(rows_v.at[bi], acc.at[dst_v.at[j]], add=True)
        return carry

    lax.fori_loop(0, K2 // NBUF, body, 0)
    plsc.subcore_barrier()
    pltpu.sync_copy(acc.at[pl.ds(s * RPT, RPT)],
                    out_hbm.at[c, pl.ds(s * RPT, RPT)])


def _tc_h1_body(x_ref, w1_ref, h1_ref):
    h1 = jnp.dot(x_ref[...], w1_ref[...],
                 preferred_element_type=jnp.float32)
    h1_ref[:N, :] = h1
    h1_ref[N:, :] = jnp.zeros((NPAD - N, D_H), jnp.float32)


def _tc_mid_body(aggp_ref, degp_ref, h1_ref, b1_ref, w2_ref,
                 g2_ref, dis_ref):
    deg = degp_ref[0, :N] + degp_ref[1, :N] + 1.0
    dis = lax.rsqrt(deg).reshape(N, 1)
    dis_ref[...] = dis
    agg = aggp_ref[0, :N, :] + aggp_ref[1, :N, :]
    t = dis * (agg + dis * h1_ref[:N, :]) + b1_ref[...]
    t = jnp.where(t > 0, t, jnp.exp(jnp.minimum(t, 0.0)) - 1.0)
    mean = jnp.mean(t, axis=0, keepdims=True)
    var = jnp.mean((t - mean) ** 2, axis=0, keepdims=True)
    t = (t - mean) * lax.rsqrt(var + 1e-5)
    t = jnp.maximum(t, 0.0)
    h2 = jnp.dot(t, w2_ref[...], preferred_element_type=jnp.float32)
    g2 = h2 * dis
    zpad = jnp.zeros((NPAD - N, DHALF), jnp.float32)
    g2_ref[0, :N, :] = g2[:, :DHALF]
    g2_ref[0, N:, :] = zpad
    g2_ref[1, :N, :] = g2[:, DHALF:]
    g2_ref[1, N:, :] = zpad


def _tc_out_body(aggp_ref, g2_ref, dis_ref, b2_ref, out_ref):
    dis = dis_ref[...]
    o0 = dis * (aggp_ref[0, :N, :] + g2_ref[0, :N, :]) + b2_ref[:DHALF]
    o1 = dis * (aggp_ref[1, :N, :] + g2_ref[1, :N, :]) + b2_ref[DHALF:]
    m = jnp.maximum(jnp.max(o0, axis=1, keepdims=True),
                    jnp.max(o1, axis=1, keepdims=True))
    se = (jnp.sum(jnp.exp(o0 - m), axis=1, keepdims=True)
          + jnp.sum(jnp.exp(o1 - m), axis=1, keepdims=True))
    lse = jnp.log(se) + m
    out_ref[:, :DHALF] = o0 - lse
    out_ref[:, DHALF:] = o1 - lse


_tc_h1 = pl.pallas_call(
    _tc_h1_body,
    out_shape=jax.ShapeDtypeStruct((NPAD, D_H), jnp.float32),
)

_tc_mid = pl.pallas_call(
    _tc_mid_body,
    out_shape=(jax.ShapeDtypeStruct((NC, NPAD, DHALF), jnp.float32),
               jax.ShapeDtypeStruct((N, 1), jnp.float32)),
)

_tc_out = pl.pallas_call(
    _tc_out_body,
    out_shape=jax.ShapeDtypeStruct((N, D_OUT), jnp.float32),
)


def kernel(x, edge_index, W1, b1, W2, b2):
    src = edge_index[0]
    dst = edge_index[1]
    pad = EPAD - E
    src_r = jnp.concatenate(
        [src, jnp.zeros((pad,), src.dtype)]).reshape(TOTC, CHUNK)
    dst_r = jnp.concatenate(
        [dst, jnp.full((pad,), N, dst.dtype)]).reshape(TOTC, CHUNK)

    h1 = _tc_h1(x, W1)
    deg_p = _deg_kernel(dst_r)
    agg1 = _scatter32(h1, deg_p, src_r, dst_r)
    g2, dis = _tc_mid(agg1, deg_p, h1, b1, W2)
    agg2 = _scatter64f(g2, src_r, dst_r)
    return _tc_out(agg2, g2, dis, b2)


# speedup vs baseline: 1.0084x; 1.0084x over previous
import functools

import jax
import jax.numpy as jnp
from jax import lax
from jax.experimental import pallas as pl
from jax.experimental.pallas import tpu as pltpu
from jax.experimental.pallas import tpu_sc as plsc

N = 10000
E = 320000
D_IN = 128
D_H = 32
D_OUT = 64

NC = 2
NS = 16
NW = NC * NS
L = 16

CHUNK = 128
NBUF = 4
K = -(-E // (NW * CHUNK * NBUF)) * NBUF
K2 = K * NC
TOTC = NW * K
EPAD = TOTC * CHUNK
NPAD = 10240
RPT = NPAD // NS
ZR = 128

_MESH = plsc.VectorSubcoreMesh(
    core_axis_name="c", subcore_axis_name="s", num_cores=NC, num_subcores=NS)


@functools.partial(
    pl.kernel,
    out_type=jax.ShapeDtypeStruct((NC, NPAD), jnp.float32),
    mesh=_MESH,
    scratch_types=[
        pltpu.VMEM((K, CHUNK), jnp.int32),
        pltpu.VMEM((CHUNK,), jnp.float32),
        pltpu.VMEM((RPT,), jnp.float32),
        pltpu.VMEM_SHARED((NPAD,), jnp.float32),
    ],
)
def _deg_kernel(dst_hbm, out_hbm, dst_v, ones_v, zrow_v, acc):
    c = lax.axis_index("c")
    s = lax.axis_index("s")
    wid = c * NS + s
    for i in range(CHUNK // L):
        ones_v[pl.ds(i * L, L)] = jnp.ones((L,), jnp.float32)
    for i in range(RPT // L):
        zrow_v[pl.ds(i * L, L)] = jnp.zeros((L,), jnp.float32)
    pltpu.sync_copy(zrow_v, acc.at[pl.ds(s * RPT, RPT)])
    pltpu.sync_copy(dst_hbm.at[pl.ds(wid * K, K)], dst_v)
    plsc.subcore_barrier()

    def body(j, carry):
        pltpu.sync_copy(ones_v, acc.at[dst_v.at[j]], add=True)
        return carry

    lax.fori_loop(0, K, body, 0)
    plsc.subcore_barrier()
    pltpu.sync_copy(acc.at[pl.ds(s * RPT, RPT)],
                    out_hbm.at[c, pl.ds(s * RPT, RPT)])


def _rsqrt16(x):
    i = lax.bitcast_convert_type(x, jnp.int32)
    i = jnp.int32(0x5F3759DF) - lax.shift_right_arithmetic(i, 1)
    y = lax.bitcast_convert_type(i, jnp.float32)
    for _ in range(3):
        y = y * (1.5 - 0.5 * x * y * y)
    return y


def _make_scatter_kernel(D):
    @functools.partial(
        pl.kernel,
        out_type=jax.ShapeDtypeStruct((NC, NPAD, D), jnp.float32),
        mesh=_MESH,
        compiler_params=pltpu.CompilerParams(use_tc_tiling_on_sc=False,
                                             needs_layout_passes=False),
        scratch_types=[
            pltpu.VMEM((K, CHUNK), jnp.int32),
            pltpu.VMEM((K, CHUNK), jnp.int32),
            pltpu.VMEM((NBUF, CHUNK, D), jnp.float32),
            pltpu.VMEM((ZR, D), jnp.float32),
            pltpu.VMEM((RPT, D), jnp.float32),
            pltpu.VMEM((RPT,), jnp.float32),
            pltpu.VMEM((RPT,), jnp.float32),
            pltpu.VMEM_SHARED((NPAD, D), jnp.float32),
            pltpu.VMEM_SHARED((NPAD, D), jnp.float32),
        ] + [pltpu.SemaphoreType.DMA] * NBUF,
    )
    def _scatter(g_hbm, degp_hbm, src_hbm, dst_hbm, out_hbm,
                 src_v, dst_v, rows_v, zb_v, hbuf, degv, disv,
                 acc, g_sh, *sems):
        c = lax.axis_index("c")
        s = lax.axis_index("s")
        wid = c * NS + s

        def zfill(i, carry):
            zrow = zb_v.at[i]
            for dj in range(D // L):
                zrow[pl.ds(dj * L, L)] = jnp.zeros((L,), jnp.float32)
            return carry

        lax.fori_loop(0, ZR, zfill, 0)
        for t in range(RPT // ZR):
            pltpu.sync_copy(zb_v, acc.at[pl.ds(s * RPT + t * ZR, ZR)])
        pltpu.sync_copy(g_hbm.at[pl.ds(s * RPT, RPT)], hbuf)
        pltpu.sync_copy(degp_hbm.at[0, pl.ds(s * RPT, RPT)], degv)
        pltpu.sync_copy(degp_hbm.at[1, pl.ds(s * RPT, RPT)], disv)

        def disgrp(i, carry):
            d = degv[pl.ds(i * L, L)] + disv[pl.ds(i * L, L)] + 1.0
            disv[pl.ds(i * L, L)] = _rsqrt16(d)
            return carry

        lax.fori_loop(0, RPT // L, disgrp, 0)

        def rowscale(r, carry):
            sc = plsc.load_gather(disv, [jnp.zeros((L,), jnp.int32) + r])
            row = hbuf.at[r]
            for dj in range(D // L):
                row[pl.ds(dj * L, L)] = row[pl.ds(dj * L, L)] * sc
            return carry

        lax.fori_loop(0, RPT, rowscale, 0)
        pltpu.sync_copy(hbuf, g_sh.at[pl.ds(s * RPT, RPT)])
        pltpu.sync_copy(src_hbm.at[pl.ds(wid * K, K)], src_v)
        pltpu.sync_copy(dst_hbm.at[pl.ds(wid * K, K)], dst_v)
        plsc.subcore_barrier()

        for b in range(NBUF - 1):
            pltpu.async_copy(g_sh.at[src_v.at[b]], rows_v.at[b], sems[b])

        def body(i, carry):
            j0 = i * NBUF
            for bi in range(NBUF):
                j = j0 + bi
                jl = j + NBUF - 1
                bl = (bi + NBUF - 1) % NBUF

                @pl.when(jl < K)
                def _():
                    pltpu.async_copy(g_sh.at[src_v.at[jl]],
                                     rows_v.at[bl], sems[bl])

                pltpu.make_async_copy(g_sh.at[src_v.at[j]],
                                      rows_v.at[bi], sems[bi]).wait()
                pltpu.sync_copy(rows_v.at[bi], acc.at[dst_v.at[j]], add=True)
            return carry

        lax.fori_loop(0, K // NBUF, body, 0)
        plsc.subcore_barrier()
        pltpu.sync_copy(acc.at[pl.ds(s * RPT, RPT)],
                        out_hbm.at[c, pl.ds(s * RPT, RPT)])

    return _scatter


_scatter32 = _make_scatter_kernel(D_H)

DHALF = D_OUT // NC


@functools.partial(
    pl.kernel,
    out_type=jax.ShapeDtypeStruct((NC, NPAD, DHALF), jnp.float32),
    mesh=_MESH,
    compiler_params=pltpu.CompilerParams(use_tc_tiling_on_sc=False),
    scratch_types=[
        pltpu.VMEM((K2, CHUNK), jnp.int32),
        pltpu.VMEM((K2, CHUNK), jnp.int32),
        pltpu.VMEM((NBUF, CHUNK, DHALF), jnp.float32),
        pltpu.VMEM((ZR, DHALF), jnp.float32),
        pltpu.VMEM_SHARED((NPAD, DHALF), jnp.float32),
        pltpu.VMEM_SHARED((NPAD, DHALF), jnp.float32),
    ] + [pltpu.SemaphoreType.DMA] * NBUF,
)
def _scatter64f(g_hbm, src_hbm, dst_hbm, out_hbm,
                src_v, dst_v, rows_v, zb_v, acc, g_sh, *sems):
    c = lax.axis_index("c")
    s = lax.axis_index("s")

    def zfill(i, carry):
        for dj in range(DHALF // L):
            zb_v[i, pl.ds(dj * L, L)] = jnp.zeros((L,), jnp.float32)
        return carry

    lax.fori_loop(0, ZR, zfill, 0)
    for t in range(RPT // ZR):
        pltpu.sync_copy(zb_v, acc.at[pl.ds(s * RPT + t * ZR, ZR)])
    pltpu.sync_copy(g_hbm.at[c, pl.ds(s * RPT, RPT)],
                    g_sh.at[pl.ds(s * RPT, RPT)])
    pltpu.sync_copy(src_hbm.at[pl.ds(s * K2, K2)], src_v)
    pltpu.sync_copy(dst_hbm.at[pl.ds(s * K2, K2)], dst_v)
    plsc.subcore_barrier()

    for b in range(NBUF - 1):
        pltpu.async_copy(g_sh.at[src_v.at[b]], rows_v.at[b], sems[b])

    def body(i, carry):
        j0 = i * NBUF
        for bi in range(NBUF):
            j = j0 + bi
            jl = j + NBUF - 1
            bl = (bi + NBUF - 1) % NBUF

            @pl.when(jl < K2)
            def _():
                pltpu.async_copy(g_sh.at[src_v.at[jl]],
                                 rows_v.at[bl], sems[bl])

            pltpu.make_async_copy(g_sh.at[src_v.at[j]],
                                  rows_v.at[bi], sems[bi]).wait()
            pltpu.sync_copy(rows_v.at[bi], acc.at[dst_v.at[j]], add=True)
        return carry

    lax.fori_loop(0, K2 // NBUF, body, 0)
    plsc.subcore_barrier()
    pltpu.sync_copy(acc.at[pl.ds(s * RPT, RPT)],
                    out_hbm.at[c, pl.ds(s * RPT, RPT)])


def _tc_h1_body(x_ref, w1_ref, h1_ref):
    h1 = jnp.dot(x_ref[...], w1_ref[...],
                 preferred_element_type=jnp.float32)
    h1_ref[:N, :] = h1
    h1_ref[N:, :] = jnp.zeros((NPAD - N, D_H), jnp.float32)


def _tc_mid_body(aggp_ref, degp_ref, h1_ref, b1_ref, w2_ref,
                 g2_ref, dis_ref):
    deg = degp_ref[0, :N] + degp_ref[1, :N] + 1.0
    dis = lax.rsqrt(deg).reshape(N, 1)
    dis_ref[...] = dis
    agg = aggp_ref[0, :N, :] + aggp_ref[1, :N, :]
    t = dis * (agg + dis * h1_ref[:N, :]) + b1_ref[...]
    t = jnp.where(t > 0, t, jnp.exp(jnp.minimum(t, 0.0)) - 1.0)
    mean = jnp.mean(t, axis=0, keepdims=True)
    var = jnp.mean((t - mean) ** 2, axis=0, keepdims=True)
    t = (t - mean) * lax.rsqrt(var + 1e-5)
    t = jnp.maximum(t, 0.0)
    h2 = jnp.dot(t, w2_ref[...], preferred_element_type=jnp.float32)
    g2 = h2 * dis
    zpad = jnp.zeros((NPAD - N, DHALF), jnp.float32)
    g2_ref[0, :N, :] = g2[:, :DHALF]
    g2_ref[0, N:, :] = zpad
    g2_ref[1, :N, :] = g2[:, DHALF:]
    g2_ref[1, N:, :] = zpad


def _tc_out_body(aggp_ref, g2_ref, dis_ref, b2_ref, out_ref):
    dis = dis_ref[...]
    o0 = dis * (aggp_ref[0, :N, :] + g2_ref[0, :N, :]) + b2_ref[:DHALF]
    o1 = dis * (aggp_ref[1, :N, :] + g2_ref[1, :N, :]) + b2_ref[DHALF:]
    m = jnp.maximum(jnp.max(o0, axis=1, keepdims=True),
                    jnp.max(o1, axis=1, keepdims=True))
    se = (jnp.sum(jnp.exp(o0 - m), axis=1, keepdims=True)
          + jnp.sum(jnp.exp(o1 - m), axis=1, keepdims=True))
    lse = jnp.log(se) + m
    out_ref[:, :DHALF] = o0 - lse
    out_ref[:, DHALF:] = o1 - lse


_tc_h1 = pl.pallas_call(
    _tc_h1_body,
    out_shape=jax.ShapeDtypeStruct((NPAD, D_H), jnp.float32),
)

_tc_mid = pl.pallas_call(
    _tc_mid_body,
    out_shape=(jax.ShapeDtypeStruct((NC, NPAD, DHALF), jnp.float32),
               jax.ShapeDtypeStruct((N, 1), jnp.float32)),
)

_tc_out = pl.pallas_call(
    _tc_out_body,
    out_shape=jax.ShapeDtypeStruct((N, D_OUT), jnp.float32),
)


def kernel(x, edge_index, W1, b1, W2, b2):
    ec = E // CHUNK
    src_r = jax.lax.dynamic_update_slice(
        jnp.zeros((TOTC, CHUNK), edge_index.dtype),
        edge_index[0].reshape(ec, CHUNK), (0, 0))
    dst_r = jax.lax.dynamic_update_slice(
        jnp.full((TOTC, CHUNK), N, edge_index.dtype),
        edge_index[1].reshape(ec, CHUNK), (0, 0))

    h1 = _tc_h1(x, W1)
    deg_p = _deg_kernel(dst_r)
    agg1 = _scatter32(h1, deg_p, src_r, dst_r)
    g2, dis = _tc_mid(agg1, deg_p, h1, b1, W2)
    agg2 = _scatter64f(g2, src_r, dst_r)
    return _tc_out(agg2, g2, dis, b2)
